# Initial kernel scaffold; baseline (speedup 1.0000x reference)
import jax
import jax.numpy as jnp
from jax.experimental import pallas as pl


def kernel(x, w1, b1, w2, b2, w25, b25, w3, b3, w4, b4, w5, b5):
    raise NotImplementedError("write your pallas kernel here")


# speedup vs baseline: 1.1633x; 1.1633x over previous
import jax
import jax.numpy as jnp
from jax.experimental import pallas as pl
from jax.experimental.pallas import tpu as pltpu


def _round_up(x, m):
    return ((x + m - 1) // m) * m


def _mlp_kernel(xt_ref, w1_ref, b1_ref, w2_ref, b2_ref,
                w25_ref, b25_ref, w3_ref, b3_ref,
                w4_ref, b4_ref, w5_ref, b5_ref, out_ref):
    xt = xt_ref[...]
    w1 = w1_ref[...]
    b1 = b1_ref[...]
    K = xt.shape[0]

    slabs = [
        jnp.maximum(w1 * xt[k:k + 1, :] + b1, 0.0).astype(jnp.bfloat16)
        for k in range(K)
    ]
    h1 = jnp.concatenate(slabs, axis=0)

    y = jnp.dot(w2_ref[...], h1, preferred_element_type=jnp.float32)
    y = jnp.maximum(y + b2_ref[...], 0.0)
    y = jnp.maximum(
        jnp.dot(w25_ref[...], y, preferred_element_type=jnp.float32)
        + b25_ref[...], 0.0)
    y = jnp.maximum(
        jnp.dot(w3_ref[...], y, preferred_element_type=jnp.float32)
        + b3_ref[...], 0.0)
    y = jnp.maximum(
        jnp.dot(w4_ref[...], y, preferred_element_type=jnp.float32)
        + b4_ref[...], 0.0)
    y = (jnp.dot(w5_ref[...], y, preferred_element_type=jnp.float32)
         + b5_ref[...])
    out_ref[...] = y.astype(out_ref.dtype)


def kernel(x, w1, b1, w2, b2, w25, b25, w3, b3, w4, b4, w5, b5):
    B, K = x.shape
    H = w1.shape[0]

    xt = x.T

    lane = 128
    tb = min(4096, _round_up(B, lane))
    padded_b = _round_up(B, tb)
    if padded_b // tb < 2 and padded_b > lane:
        tb = _round_up(pl.cdiv(padded_b, 2), lane)
        padded_b = tb * pl.cdiv(padded_b, tb)
    if padded_b != B:
        xt = jnp.pad(xt, ((0, 0), (0, padded_b - B)))
    grid = (padded_b // tb,)

    def col(v):
        return v.reshape(-1, 1)

    args = (xt, w1.reshape(H, 1), col(b1),
            w2.astype(jnp.bfloat16), col(b2),
            w25, col(b25),
            w3, col(b3),
            w4, col(b4),
            w5, col(b5))

    in_specs = [pl.BlockSpec((K, tb), lambda i: (0, i))]
    in_specs += [pl.BlockSpec(a.shape, lambda i: (0, 0),
                              pipeline_mode=pl.Buffered(1))
                 for a in args[1:]]

    out = pl.pallas_call(
        _mlp_kernel,
        out_shape=jax.ShapeDtypeStruct((1, padded_b), x.dtype),
        grid=grid,
        in_specs=in_specs,
        out_specs=pl.BlockSpec((1, tb), lambda i: (0, i)),
        compiler_params=pltpu.CompilerParams(
            dimension_semantics=("parallel",),
            vmem_limit_bytes=64 * 1024 * 1024),
    )(*args)
    return out[0, :B].reshape(B, 1)
